# baseline (speedup 1.0000x reference)
import jax
import jax.numpy as jnp
from jax import lax
from jax.experimental import pallas as pl
from jax.experimental.pallas import tpu as pltpu

B = 16
L = 4096
D = 256
S = 32
BS = B * S
NROW = 16
NSEM = 16


def _resample_tc(table, lens_row, out, pidx_v, pidx_s, win, sem_i, sem,
                 sem2):
    t = lax.broadcasted_iota(jnp.int32, (BS, 1), 0)
    b_id = t & (B - 1)
    k_id = t >> 4
    onehot = b_id == lax.broadcasted_iota(jnp.int32, (BS, B), 1)
    l_row = jnp.broadcast_to(lens_row[...], (BS, B))
    l = jnp.sum(jnp.where(onehot, l_row, 0), axis=1, keepdims=True)

    l_f = l.astype(jnp.float32)
    g = jnp.minimum(l & (-l), S)
    g_f = g.astype(jnp.float32)
    step_f = l_f / g_f
    j_f = k_id.astype(jnp.float32) * step_f
    scale = g_f * (1.0 / S)
    pos = (j_f + 0.5) * scale - 0.5
    pos = jnp.minimum(jnp.maximum(pos, 0.0), l_f - 1.0)
    lo = pos.astype(jnp.int32)
    hi = jnp.minimum(lo + 1, l - 1)
    w = pos - lo.astype(jnp.float32)

    row0 = b_id * L
    lo_g = row0 + lo
    hi_g = row0 + hi
    base = (lo_g >> 3) << 3
    u = lo_g - base
    v = hi_g - base
    bad = (v == 8).astype(jnp.int32)
    pidx_v[0:BS, :] = jnp.concatenate([base, bad], axis=1)
    nbad = jnp.broadcast_to(jnp.sum(bad, axis=0, keepdims=True), (8, 2))
    pidx_v[BS:BS + 8, :] = nbad

    r_iota = lax.broadcasted_iota(jnp.int32, (BS, NROW), 1)
    C = (jnp.where(u == r_iota, 1.0 - w, 0.0)
         + jnp.where(v == r_iota, w, 0.0))

    idx_cp = pltpu.make_async_copy(pidx_v, pidx_s, sem_i)
    idx_cp.start()
    idx_cp.wait()

    copies = []
    for tt in range(BS):
        bs = pl.multiple_of(pidx_s[tt, 0], 8)
        cp = pltpu.make_async_copy(
            table.at[pl.ds(bs, 8), :],
            win.at[tt, pl.ds(0, 8), :],
            sem.at[tt % NSEM],
        )
        cp.start()
        copies.append(cp)

        @pl.when(pidx_s[tt, 1] == 1)
        def _(bs=bs, tt=tt):
            pltpu.make_async_copy(
                table.at[pl.ds(bs + 8, 8), :],
                win.at[tt, pl.ds(8, 8), :],
                sem2,
            ).start()

    def _drain(_, carry):
        pltpu.make_async_copy(
            table.at[pl.ds(0, 8), :],
            win.at[0, pl.ds(8, 8), :],
            sem2,
        ).wait()
        return carry

    lax.fori_loop(0, pidx_s[BS, 0], _drain, 0)
    GRP = 64
    for g in range(BS // GRP):
        t0 = g * GRP
        for cp in copies[t0:t0 + GRP]:
            cp.wait()
        acc = C[t0:t0 + GRP, 0:1] * win[pl.ds(t0, GRP), 0, :]
        for r in range(1, 9):
            cr = C[t0:t0 + GRP, r:r + 1]
            term = cr * win[pl.ds(t0, GRP), r, :]
            if r == 8:
                term = jnp.where(cr != 0.0, term, 0.0)
            acc = acc + term
        for kk in range(GRP // B):
            k = g * (GRP // B) + kk
            out[:, pl.ds(k * D, D)] = acc[kk * B:(kk + 1) * B, :]
    out[:, pl.ds(S * D, 1)] = l_f[0:B, :]


def kernel(padded_input, lengths):
    table = padded_input.reshape(B * L, D)
    lens_row = lengths.astype(jnp.int32).reshape(1, B)
    return pl.pallas_call(
        _resample_tc,
        in_specs=[
            pl.BlockSpec(memory_space=pltpu.MemorySpace.HBM),
            pl.BlockSpec(memory_space=pltpu.MemorySpace.VMEM),
        ],
        out_specs=pl.BlockSpec(memory_space=pltpu.MemorySpace.VMEM),
        out_shape=jax.ShapeDtypeStruct((B, S * D + 1), jnp.float32),
        scratch_shapes=[
            pltpu.VMEM((BS + 8, 2), jnp.int32),
            pltpu.SMEM((BS + 8, 2), jnp.int32),
            pltpu.VMEM((BS, NROW, D), jnp.float32),
            pltpu.SemaphoreType.DMA,
            pltpu.SemaphoreType.DMA((NSEM,)),
            pltpu.SemaphoreType.DMA,
        ],
    )(table, lens_row)


# speedup vs baseline: 1.0510x; 1.0510x over previous
import jax
import jax.numpy as jnp
from jax import lax
from jax.experimental import pallas as pl
from jax.experimental.pallas import tpu as pltpu

B = 16
L = 4096
D = 256
S = 32
BS = B * S
NROW = 16
NSEM = 32


def _resample_tc(table, lens_row, out, pidx_v, pidx_s, win, sem_i, sem):
    t = lax.broadcasted_iota(jnp.int32, (BS, 1), 0)
    b_id = t & (B - 1)
    k_id = t >> 4
    onehot = b_id == lax.broadcasted_iota(jnp.int32, (BS, B), 1)
    l_row = jnp.broadcast_to(lens_row[...], (BS, B))
    l = jnp.sum(jnp.where(onehot, l_row, 0), axis=1, keepdims=True)

    l_f = l.astype(jnp.float32)
    g = jnp.minimum(l & (-l), S)
    g_f = g.astype(jnp.float32)
    step_f = l_f / g_f
    j_f = k_id.astype(jnp.float32) * step_f
    scale = g_f * (1.0 / S)
    pos = (j_f + 0.5) * scale - 0.5
    pos = jnp.minimum(jnp.maximum(pos, 0.0), l_f - 1.0)
    lo = pos.astype(jnp.int32)
    hi = jnp.minimum(lo + 1, l - 1)
    w = pos - lo.astype(jnp.float32)

    row0 = b_id * L
    lo_g = row0 + lo
    hi_g = row0 + hi
    p = jnp.maximum(jnp.minimum(lo, l - 2), 0) + row0
    base = jnp.minimum((p >> 3) << 3, B * L - NROW)
    u = lo_g - base
    v = hi_g - base
    pidx_v[...] = base

    r_iota = lax.broadcasted_iota(jnp.int32, (BS, NROW), 1)
    C = (jnp.where(u == r_iota, 1.0 - w, 0.0)
         + jnp.where(v == r_iota, w, 0.0))

    idx_cp = pltpu.make_async_copy(pidx_v, pidx_s, sem_i)
    idx_cp.start()
    idx_cp.wait()

    copies = []
    for tt in range(BS):
        bs = pl.multiple_of(pidx_s[tt, 0], 8)
        cp = pltpu.make_async_copy(
            table.at[pl.ds(bs, NROW), :],
            win.at[tt],
            sem.at[tt % NSEM],
        )
        cp.start()
        copies.append(cp)
    GRP = 32
    for g in range(BS // GRP):
        t0 = g * GRP
        for cp in copies[t0:t0 + GRP]:
            cp.wait()
        acc = C[t0:t0 + GRP, 0:1] * win[pl.ds(t0, GRP), 0, :]
        for r in range(1, NROW):
            acc = acc + C[t0:t0 + GRP, r:r + 1] * win[pl.ds(t0, GRP), r, :]
        for kk in range(GRP // B):
            k = g * (GRP // B) + kk
            out[:, pl.ds(k * D, D)] = acc[kk * B:(kk + 1) * B, :]
    out[:, pl.ds(S * D, 1)] = l_f[0:B, :]


def kernel(padded_input, lengths):
    table = padded_input.reshape(B * L, D)
    lens_row = lengths.astype(jnp.int32).reshape(1, B)
    return pl.pallas_call(
        _resample_tc,
        in_specs=[
            pl.BlockSpec(memory_space=pltpu.MemorySpace.HBM),
            pl.BlockSpec(memory_space=pltpu.MemorySpace.VMEM),
        ],
        out_specs=pl.BlockSpec(memory_space=pltpu.MemorySpace.VMEM),
        out_shape=jax.ShapeDtypeStruct((B, S * D + 1), jnp.float32),
        scratch_shapes=[
            pltpu.VMEM((BS, 1), jnp.int32),
            pltpu.SMEM((BS, 1), jnp.int32),
            pltpu.VMEM((BS, NROW, D), jnp.float32),
            pltpu.SemaphoreType.DMA,
            pltpu.SemaphoreType.DMA((NSEM,)),
        ],
    )(table, lens_row)
